# baseline (speedup 1.0000x reference)
import jax
import jax.numpy as jnp
from jax.experimental import pallas as pl
from jax.experimental.pallas import tpu as pltpu

_B = 32
_A = 3
_ATTR = 85
_GW = 76
_S = _GW * _GW
_STRIDE = 8.0
_ANCH_W = (116.0, 156.0, 373.0)
_ANCH_H = (90.0, 198.0, 326.0)


def _decode_block(x_ref, o_ref):
    gx = jax.lax.broadcasted_iota(jnp.int32, (1, 1, _GW), 2).astype(jnp.float32)
    gy = jax.lax.broadcasted_iota(jnp.int32, (1, _GW, 1), 1).astype(jnp.float32)
    for a in range(_A):
        v = x_ref[0, _ATTR * a:_ATTR * (a + 1)]
        sig = jax.nn.sigmoid(v)
        row0 = (sig[0:1] + gx) * _STRIDE
        row1 = (sig[1:2] + gy) * _STRIDE
        row2 = jnp.exp(v[2:3]) * _ANCH_W[a]
        row3 = jnp.exp(v[3:4]) * _ANCH_H[a]
        t = jnp.concatenate([row0, row1, row2, row3, sig[4:]], axis=0)
        o_ref[0, a] = jnp.transpose(t, (1, 2, 0))


def kernel(inputs):
    out5 = pl.pallas_call(
        _decode_block,
        grid=(_B,),
        in_specs=[pl.BlockSpec((1, _A * _ATTR, _GW, _GW), lambda b: (b, 0, 0, 0))],
        out_specs=pl.BlockSpec((1, _A, _GW, _GW, _ATTR), lambda b: (b, 0, 0, 0, 0)),
        out_shape=jax.ShapeDtypeStruct((_B, _A, _GW, _GW, _ATTR), jnp.float32),
        compiler_params=pltpu.CompilerParams(
            dimension_semantics=("parallel",),
        ),
    )(inputs)
    return out5.reshape(_B, _A * _S, _ATTR)
